# baseline (speedup 1.0000x reference)
import functools

import jax
import jax.numpy as jnp
from jax import lax
from jax.experimental import pallas as pl
from jax.experimental.pallas import tpu as pltpu
from jax.experimental.pallas import tpu_sc as plsc

N = 10000
E = 320000
IN_CH = 128
OUT_CH = 64
CAT = 2 * OUT_CH
HALF = 64

NC = 2
NS = 16
NW = NC * NS
EPT = E // NW
K = 128
NCHUNK = 80
EPT_PAD = NCHUNK * K
E_PAD = NW * EPT_PAD
NG2 = NCHUNK // 2
N_PAD = 10240
RPT = N_PAD // NS
RCH = 160
NRCH = RPT // RCH


def _sc_mesh():
    return plsc.VectorSubcoreMesh(core_axis_name="c", subcore_axis_name="s")


def _deg_sc(dst_idx, ones_blk, zeros_blk):
    @functools.partial(
        pl.kernel,
        out_type=jax.ShapeDtypeStruct((NC, NS, RPT, 16), jnp.float32),
        mesh=_sc_mesh(),
        compiler_params=pltpu.CompilerParams(use_tc_tiling_on_sc=False),
        scratch_types=[
            pltpu.VMEM((NCHUNK, K), jnp.int32),
            pltpu.VMEM((K, 16), jnp.float32),
            pltpu.VMEM((RCH, 16), jnp.float32),
            pltpu.VMEM_SHARED((N_PAD, 16), jnp.float32),
        ],
    )
    def body(dst_hbm, ones_hbm, zeros_hbm, out_hbm, idx_v, ones_v, buf_v, acc_sh):
        c = lax.axis_index("c")
        s = lax.axis_index("s")
        w = c * NS + s
        pltpu.sync_copy(zeros_hbm, buf_v)
        for i in range(NRCH):
            pltpu.sync_copy(buf_v, acc_sh.at[pl.ds(s * RPT + i * RCH, RCH)])
        pltpu.sync_copy(ones_hbm, ones_v)
        pltpu.sync_copy(dst_hbm.at[w], idx_v)
        plsc.subcore_barrier()

        def chunk(j, carry):
            pltpu.sync_copy(ones_v, acc_sh.at[idx_v.at[j]], add=True)
            return carry

        lax.fori_loop(0, NCHUNK, chunk, 0)
        plsc.subcore_barrier()
        for i in range(NRCH):
            pltpu.sync_copy(acc_sh.at[pl.ds(s * RPT + i * RCH, RCH)], buf_v)
            pltpu.sync_copy(buf_v, out_hbm.at[c, s, pl.ds(i * RCH, RCH)])

    return body(dst_idx, ones_blk, zeros_blk)


def _agg_sc(hpA, hpB, src_idx, dst_idx, zeros_blk):
    @functools.partial(
        pl.kernel,
        out_type=jax.ShapeDtypeStruct((2, NC, NS, RPT, HALF), jnp.float32),
        mesh=_sc_mesh(),
        compiler_params=pltpu.CompilerParams(use_tc_tiling_on_sc=False),
        scratch_types=[
            pltpu.VMEM((NCHUNK, K), jnp.int32),
            pltpu.VMEM((NCHUNK, K), jnp.int32),
            pltpu.VMEM((K, HALF), jnp.float32),
            pltpu.VMEM((K, HALF), jnp.float32),
            pltpu.VMEM((RCH, HALF), jnp.float32),
            pltpu.VMEM_SHARED((N_PAD, HALF), jnp.float32),
            pltpu.SemaphoreType.DMA,
            pltpu.SemaphoreType.DMA,
        ],
    )
    def body(hpA_hbm, hpB_hbm, src_hbm, dst_hbm, zeros_hbm, out_hbm,
             sidx_v, didx_v, rowsA_v, rowsB_v, buf_v, acc_sh, semA, semB):
        c = lax.axis_index("c")
        s = lax.axis_index("s")
        w = c * NS + s
        pltpu.sync_copy(src_hbm.at[w], sidx_v)
        pltpu.sync_copy(dst_hbm.at[w], didx_v)

        for half, hp_hbm in ((0, hpA_hbm), (1, hpB_hbm)):
            pltpu.sync_copy(zeros_hbm, buf_v)
            for i in range(NRCH):
                pltpu.sync_copy(buf_v, acc_sh.at[pl.ds(s * RPT + i * RCH, RCH)])
            plsc.subcore_barrier()

            def chunk(j, carry):
                pltpu.async_copy(hp_hbm.at[sidx_v.at[j]], rowsA_v, semA).wait()
                pltpu.sync_copy(rowsA_v, acc_sh.at[didx_v.at[j]], add=True)
                return carry

            lax.fori_loop(0, NCHUNK, chunk, 0)
            plsc.subcore_barrier()
            for i in range(NRCH):
                pltpu.sync_copy(acc_sh.at[pl.ds(s * RPT + i * RCH, RCH)], buf_v)
                pltpu.sync_copy(buf_v, out_hbm.at[half, c, s, pl.ds(i * RCH, RCH)])
            plsc.subcore_barrier()

    return body(hpA, hpB, src_idx, dst_idx, zeros_blk)


_BLK = 1000
_GRID = N // _BLK


def _mm_tc(x, W):
    def body(x_ref, w_ref, o_ref):
        o_ref[...] = jnp.dot(x_ref[...], w_ref[...],
                             preferred_element_type=jnp.float32)

    return pl.pallas_call(
        body,
        grid=(_GRID,),
        in_specs=[
            pl.BlockSpec((_BLK, x.shape[1]), lambda i: (i, 0)),
            pl.BlockSpec(W.shape, lambda i: (0, 0)),
        ],
        out_specs=pl.BlockSpec((_BLK, W.shape[1]), lambda i: (i, 0)),
        out_shape=jax.ShapeDtypeStruct((N, W.shape[1]), jnp.float32),
    )(x, W)


def _dis_hp_tc(deg0, deg1, m1):
    def body(d0_ref, d1_ref, m_ref, hpA_ref, hpB_ref, dis_ref):
        deg = d0_ref[:, :1] + d1_ref[:, :1] + 1.0
        dis = lax.rsqrt(deg)
        dis_ref[...] = dis
        hp = dis * m_ref[...]
        hpA_ref[...] = hp[:, :HALF]
        hpB_ref[...] = hp[:, HALF:]

    return pl.pallas_call(
        body,
        grid=(_GRID,),
        in_specs=[
            pl.BlockSpec((_BLK, 16), lambda i: (i, 0)),
            pl.BlockSpec((_BLK, 16), lambda i: (i, 0)),
            pl.BlockSpec((_BLK, CAT), lambda i: (i, 0)),
        ],
        out_specs=[
            pl.BlockSpec((_BLK, HALF), lambda i: (i, 0)),
            pl.BlockSpec((_BLK, HALF), lambda i: (i, 0)),
            pl.BlockSpec((_BLK, 1), lambda i: (i, 0)),
        ],
        out_shape=[
            jax.ShapeDtypeStruct((N, HALF), jnp.float32),
            jax.ShapeDtypeStruct((N, HALF), jnp.float32),
            jax.ShapeDtypeStruct((N, 1), jnp.float32),
        ],
    )(deg0, deg1, m1)


def _mid_tc(p0A, p1A, p0B, p1B, hp1A, hp1B, dis, b1, WcatA, WcatB):
    def body(p0A_ref, p1A_ref, p0B_ref, p1B_ref, hpA_ref, hpB_ref,
             dis_ref, b_ref, wA_ref, wB_ref, oA_ref, oB_ref):
        dis = dis_ref[...]
        tA = dis * (p0A_ref[...] + p1A_ref[...] + hpA_ref[...]) + b_ref[:, :HALF]
        tB = dis * (p0B_ref[...] + p1B_ref[...] + hpB_ref[...]) + b_ref[:, HALF:]
        hA = jnp.maximum(tA, 0.0)
        hB = jnp.maximum(tB, 0.0)
        nrm = jnp.sqrt(jnp.sum(hA * hA, axis=1, keepdims=True)
                       + jnp.sum(hB * hB, axis=1, keepdims=True))
        inv = 1.0 / jnp.maximum(nrm, 1e-12)
        hA = hA * inv
        hB = hB * inv
        hA = jnp.maximum(hA, 0.0)
        hB = jnp.maximum(hB, 0.0)
        m2 = (jnp.dot(hA, wA_ref[...], preferred_element_type=jnp.float32)
              + jnp.dot(hB, wB_ref[...], preferred_element_type=jnp.float32))
        hp2 = dis * m2
        oA_ref[...] = hp2[:, :HALF]
        oB_ref[...] = hp2[:, HALF:]

    blk = lambda d: pl.BlockSpec((_BLK, d), lambda i: (i, 0))
    return pl.pallas_call(
        body,
        grid=(_GRID,),
        in_specs=[
            blk(HALF), blk(HALF), blk(HALF), blk(HALF), blk(HALF), blk(HALF),
            blk(1),
            pl.BlockSpec((1, CAT), lambda i: (0, 0)),
            pl.BlockSpec((HALF, CAT), lambda i: (0, 0)),
            pl.BlockSpec((HALF, CAT), lambda i: (0, 0)),
        ],
        out_specs=[blk(HALF), blk(HALF)],
        out_shape=[
            jax.ShapeDtypeStruct((N, HALF), jnp.float32),
            jax.ShapeDtypeStruct((N, HALF), jnp.float32),
        ],
    )(p0A, p1A, p0B, p1B, hp1A, hp1B, dis, b1, WcatA, WcatB)


def _final_tc(p0A, p1A, p0B, p1B, hp2A, hp2B, dis, bmu, bls):
    def body(p0A_ref, p1A_ref, p0B_ref, p1B_ref, hpA_ref, hpB_ref,
             dis_ref, bmu_ref, bls_ref, mu_ref, ls_ref):
        dis = dis_ref[...]
        mu_ref[...] = dis * (p0A_ref[...] + p1A_ref[...] + hpA_ref[...]) \
            + bmu_ref[...]
        ls_ref[...] = dis * (p0B_ref[...] + p1B_ref[...] + hpB_ref[...]) \
            + bls_ref[...]

    blk = lambda d: pl.BlockSpec((_BLK, d), lambda i: (i, 0))
    return pl.pallas_call(
        body,
        grid=(_GRID,),
        in_specs=[
            blk(HALF), blk(HALF), blk(HALF), blk(HALF), blk(HALF), blk(HALF),
            blk(1),
            pl.BlockSpec((1, HALF), lambda i: (0, 0)),
            pl.BlockSpec((1, HALF), lambda i: (0, 0)),
        ],
        out_specs=[blk(HALF), blk(HALF)],
        out_shape=[
            jax.ShapeDtypeStruct((N, HALF), jnp.float32),
            jax.ShapeDtypeStruct((N, HALF), jnp.float32),
        ],
    )(p0A, p1A, p0B, p1B, hp2A, hp2B, dis, bmu, bls)


def kernel(x, edge_index, W1, b1, Wmu, bmu, Wls, bls):
    n_fake = E_PAD - E
    fake_src = jnp.zeros((n_fake,), jnp.int32)
    fake_dst = N + (jnp.arange(n_fake, dtype=jnp.int32) % (N_PAD - N))
    src = jnp.concatenate([edge_index[0].astype(jnp.int32), fake_src])
    dst = jnp.concatenate([edge_index[1].astype(jnp.int32), fake_dst])
    src = src.reshape(NW, NCHUNK, K)
    dst = dst.reshape(NW, NCHUNK, K)

    ones_blk = jnp.ones((K, 16), jnp.float32)
    zeros16 = jnp.zeros((RCH, 16), jnp.float32)
    zeros64 = jnp.zeros((RCH, HALF), jnp.float32)

    Wcat = jnp.concatenate([Wmu, Wls], axis=1)
    b1r = b1.reshape(1, CAT)
    bmur = bmu.reshape(1, HALF)
    blsr = bls.reshape(1, HALF)

    deg_parts = _deg_sc(dst, ones_blk, zeros16).reshape(NC, N_PAD, 16)[:, :N]
    m1 = _mm_tc(x, W1)
    hp1A, hp1B, dis = _dis_hp_tc(deg_parts[0], deg_parts[1], m1)

    s1 = _agg_sc(hp1A, hp1B, src, dst, zeros64).reshape(2, NC, N_PAD, HALF)[:, :, :N]
    hp2A, hp2B = _mid_tc(s1[0, 0], s1[0, 1], s1[1, 0], s1[1, 1],
                         hp1A, hp1B, dis, b1r, Wcat[:HALF], Wcat[HALF:])

    s2 = _agg_sc(hp2A, hp2B, src, dst, zeros64).reshape(2, NC, N_PAD, HALF)[:, :, :N]
    mu, logstd = _final_tc(s2[0, 0], s2[0, 1], s2[1, 0], s2[1, 1],
                           hp2A, hp2B, dis, bmur, blsr)
    return (mu, logstd)


# speedup vs baseline: 2.6390x; 2.6390x over previous
import functools

import jax
import jax.numpy as jnp
from jax import lax
from jax.experimental import pallas as pl
from jax.experimental.pallas import tpu as pltpu
from jax.experimental.pallas import tpu_sc as plsc

N = 10000
E = 320000
IN_CH = 128
OUT_CH = 64
CAT = 2 * OUT_CH
HALF = 64

NC = 2
NS = 16
NW = NC * NS
EPT = E // NW
K = 80
NCHUNK = EPT // K
NG2 = NCHUNK // 2
N_PAD = 10240
RPT = N_PAD // NS
RCH = 160
NRCH = RPT // RCH


def _sc_mesh():
    return plsc.VectorSubcoreMesh(core_axis_name="c", subcore_axis_name="s")


def _deg_sc(dst_idx, ones_blk, zeros_blk):
    @functools.partial(
        pl.kernel,
        out_type=jax.ShapeDtypeStruct((NC, NS, RPT, 16), jnp.float32),
        mesh=_sc_mesh(),
        compiler_params=pltpu.CompilerParams(use_tc_tiling_on_sc=False),
        scratch_types=[
            pltpu.VMEM((NCHUNK, K), jnp.int32),
            pltpu.VMEM((K, 16), jnp.float32),
            pltpu.VMEM((RCH, 16), jnp.float32),
            pltpu.VMEM_SHARED((N_PAD, 16), jnp.float32),
        ],
    )
    def body(dst_hbm, ones_hbm, zeros_hbm, out_hbm, idx_v, ones_v, buf_v, acc_sh):
        c = lax.axis_index("c")
        s = lax.axis_index("s")
        w = c * NS + s
        pltpu.sync_copy(zeros_hbm, buf_v)
        for i in range(NRCH):
            pltpu.sync_copy(buf_v, acc_sh.at[pl.ds(s * RPT + i * RCH, RCH)])
        pltpu.sync_copy(ones_hbm, ones_v)
        pltpu.sync_copy(dst_hbm.at[w], idx_v)
        plsc.subcore_barrier()

        def chunk(j, carry):
            pltpu.sync_copy(ones_v, acc_sh.at[idx_v.at[j]], add=True)
            return carry

        lax.fori_loop(0, NCHUNK, chunk, 0)
        plsc.subcore_barrier()
        for i in range(NRCH):
            pltpu.sync_copy(acc_sh.at[pl.ds(s * RPT + i * RCH, RCH)], buf_v)
            pltpu.sync_copy(buf_v, out_hbm.at[c, s, pl.ds(i * RCH, RCH)])

    return body(dst_idx, ones_blk, zeros_blk)


def _agg_sc(hpA, hpB, src_idx, dst_idx, zeros_blk):
    @functools.partial(
        pl.kernel,
        out_type=jax.ShapeDtypeStruct((2, NC, NS, RPT, HALF), jnp.float32),
        mesh=_sc_mesh(),
        compiler_params=pltpu.CompilerParams(use_tc_tiling_on_sc=False),
        scratch_types=[
            pltpu.VMEM((NCHUNK, K), jnp.int32),
            pltpu.VMEM((NCHUNK, K), jnp.int32),
            pltpu.VMEM((K, HALF), jnp.float32),
            pltpu.VMEM((K, HALF), jnp.float32),
            pltpu.VMEM((RCH, HALF), jnp.float32),
            pltpu.VMEM_SHARED((N_PAD, HALF), jnp.float32),
            pltpu.SemaphoreType.DMA,
            pltpu.SemaphoreType.DMA,
        ],
    )
    def body(hpA_hbm, hpB_hbm, src_hbm, dst_hbm, zeros_hbm, out_hbm,
             sidx_v, didx_v, rowsA_v, rowsB_v, buf_v, acc_sh, semA, semB):
        c = lax.axis_index("c")
        s = lax.axis_index("s")
        w = c * NS + s
        pltpu.sync_copy(src_hbm.at[w], sidx_v)
        pltpu.sync_copy(dst_hbm.at[w], didx_v)

        for half, hp_hbm in ((0, hpA_hbm), (1, hpB_hbm)):
            pltpu.sync_copy(zeros_hbm, buf_v)
            for i in range(NRCH):
                pltpu.sync_copy(buf_v, acc_sh.at[pl.ds(s * RPT + i * RCH, RCH)])
            plsc.subcore_barrier()

            pltpu.async_copy(hp_hbm.at[sidx_v.at[0]], rowsA_v, semA)

            def pair(p, carry):
                j0 = 2 * p
                j1 = j0 + 1
                pltpu.async_copy(hp_hbm.at[sidx_v.at[j1]], rowsB_v, semB)
                pltpu.make_async_copy(hp_hbm.at[sidx_v.at[j0]], rowsA_v,
                                      semA).wait()
                pltpu.sync_copy(rowsA_v, acc_sh.at[didx_v.at[j0]], add=True)
                pltpu.async_copy(hp_hbm.at[sidx_v.at[j0 + 2]], rowsA_v, semA)
                pltpu.make_async_copy(hp_hbm.at[sidx_v.at[j1]], rowsB_v,
                                      semB).wait()
                pltpu.sync_copy(rowsB_v, acc_sh.at[didx_v.at[j1]], add=True)
                return carry

            lax.fori_loop(0, NG2, pair, 0)
            pltpu.make_async_copy(hp_hbm.at[sidx_v.at[NCHUNK - 1]], rowsA_v,
                                  semA).wait()
            pltpu.sync_copy(rowsA_v, acc_sh.at[didx_v.at[NCHUNK - 1]],
                            add=True)
            plsc.subcore_barrier()
            for i in range(NRCH):
                pltpu.sync_copy(acc_sh.at[pl.ds(s * RPT + i * RCH, RCH)], buf_v)
                pltpu.sync_copy(buf_v, out_hbm.at[half, c, s, pl.ds(i * RCH, RCH)])
            plsc.subcore_barrier()

    return body(hpA, hpB, src_idx, dst_idx, zeros_blk)


_BLK = 1000
_GRID = N // _BLK


def _mm_tc(x, W):
    def body(x_ref, w_ref, o_ref):
        o_ref[...] = jnp.dot(x_ref[...], w_ref[...],
                             preferred_element_type=jnp.float32)

    return pl.pallas_call(
        body,
        grid=(_GRID,),
        in_specs=[
            pl.BlockSpec((_BLK, x.shape[1]), lambda i: (i, 0)),
            pl.BlockSpec(W.shape, lambda i: (0, 0)),
        ],
        out_specs=pl.BlockSpec((_BLK, W.shape[1]), lambda i: (i, 0)),
        out_shape=jax.ShapeDtypeStruct((N, W.shape[1]), jnp.float32),
    )(x, W)


def _dis_hp_tc(deg0, deg1, m1):
    def body(d0_ref, d1_ref, m_ref, hpA_ref, hpB_ref, dis_ref):
        deg = d0_ref[:, :1] + d1_ref[:, :1] + 1.0
        dis = lax.rsqrt(deg)
        dis_ref[...] = dis
        hp = dis * m_ref[...]
        hpA_ref[...] = hp[:, :HALF]
        hpB_ref[...] = hp[:, HALF:]

    return pl.pallas_call(
        body,
        grid=(_GRID,),
        in_specs=[
            pl.BlockSpec((_BLK, 16), lambda i: (i, 0)),
            pl.BlockSpec((_BLK, 16), lambda i: (i, 0)),
            pl.BlockSpec((_BLK, CAT), lambda i: (i, 0)),
        ],
        out_specs=[
            pl.BlockSpec((_BLK, HALF), lambda i: (i, 0)),
            pl.BlockSpec((_BLK, HALF), lambda i: (i, 0)),
            pl.BlockSpec((_BLK, 1), lambda i: (i, 0)),
        ],
        out_shape=[
            jax.ShapeDtypeStruct((N, HALF), jnp.float32),
            jax.ShapeDtypeStruct((N, HALF), jnp.float32),
            jax.ShapeDtypeStruct((N, 1), jnp.float32),
        ],
    )(deg0, deg1, m1)


def _mid_tc(p0A, p1A, p0B, p1B, hp1A, hp1B, dis, b1, WcatA, WcatB):
    def body(p0A_ref, p1A_ref, p0B_ref, p1B_ref, hpA_ref, hpB_ref,
             dis_ref, b_ref, wA_ref, wB_ref, oA_ref, oB_ref):
        dis = dis_ref[...]
        tA = dis * (p0A_ref[...] + p1A_ref[...] + hpA_ref[...]) + b_ref[:, :HALF]
        tB = dis * (p0B_ref[...] + p1B_ref[...] + hpB_ref[...]) + b_ref[:, HALF:]
        hA = jnp.maximum(tA, 0.0)
        hB = jnp.maximum(tB, 0.0)
        nrm = jnp.sqrt(jnp.sum(hA * hA, axis=1, keepdims=True)
                       + jnp.sum(hB * hB, axis=1, keepdims=True))
        inv = 1.0 / jnp.maximum(nrm, 1e-12)
        hA = hA * inv
        hB = hB * inv
        hA = jnp.maximum(hA, 0.0)
        hB = jnp.maximum(hB, 0.0)
        m2 = (jnp.dot(hA, wA_ref[...], preferred_element_type=jnp.float32)
              + jnp.dot(hB, wB_ref[...], preferred_element_type=jnp.float32))
        hp2 = dis * m2
        oA_ref[...] = hp2[:, :HALF]
        oB_ref[...] = hp2[:, HALF:]

    blk = lambda d: pl.BlockSpec((_BLK, d), lambda i: (i, 0))
    return pl.pallas_call(
        body,
        grid=(_GRID,),
        in_specs=[
            blk(HALF), blk(HALF), blk(HALF), blk(HALF), blk(HALF), blk(HALF),
            blk(1),
            pl.BlockSpec((1, CAT), lambda i: (0, 0)),
            pl.BlockSpec((HALF, CAT), lambda i: (0, 0)),
            pl.BlockSpec((HALF, CAT), lambda i: (0, 0)),
        ],
        out_specs=[blk(HALF), blk(HALF)],
        out_shape=[
            jax.ShapeDtypeStruct((N, HALF), jnp.float32),
            jax.ShapeDtypeStruct((N, HALF), jnp.float32),
        ],
    )(p0A, p1A, p0B, p1B, hp1A, hp1B, dis, b1, WcatA, WcatB)


def _final_tc(p0A, p1A, p0B, p1B, hp2A, hp2B, dis, bmu, bls):
    def body(p0A_ref, p1A_ref, p0B_ref, p1B_ref, hpA_ref, hpB_ref,
             dis_ref, bmu_ref, bls_ref, mu_ref, ls_ref):
        dis = dis_ref[...]
        mu_ref[...] = dis * (p0A_ref[...] + p1A_ref[...] + hpA_ref[...]) \
            + bmu_ref[...]
        ls_ref[...] = dis * (p0B_ref[...] + p1B_ref[...] + hpB_ref[...]) \
            + bls_ref[...]

    blk = lambda d: pl.BlockSpec((_BLK, d), lambda i: (i, 0))
    return pl.pallas_call(
        body,
        grid=(_GRID,),
        in_specs=[
            blk(HALF), blk(HALF), blk(HALF), blk(HALF), blk(HALF), blk(HALF),
            blk(1),
            pl.BlockSpec((1, HALF), lambda i: (0, 0)),
            pl.BlockSpec((1, HALF), lambda i: (0, 0)),
        ],
        out_specs=[blk(HALF), blk(HALF)],
        out_shape=[
            jax.ShapeDtypeStruct((N, HALF), jnp.float32),
            jax.ShapeDtypeStruct((N, HALF), jnp.float32),
        ],
    )(p0A, p1A, p0B, p1B, hp2A, hp2B, dis, bmu, bls)


def kernel(x, edge_index, W1, b1, Wmu, bmu, Wls, bls):
    src = edge_index[0].astype(jnp.int32).reshape(NW, NCHUNK, K)
    dst = edge_index[1].astype(jnp.int32).reshape(NW, NCHUNK, K)

    ones_blk = jnp.ones((K, 16), jnp.float32)
    zeros16 = jnp.zeros((RCH, 16), jnp.float32)
    zeros64 = jnp.zeros((RCH, HALF), jnp.float32)

    Wcat = jnp.concatenate([Wmu, Wls], axis=1)
    b1r = b1.reshape(1, CAT)
    bmur = bmu.reshape(1, HALF)
    blsr = bls.reshape(1, HALF)

    deg_parts = _deg_sc(dst, ones_blk, zeros16).reshape(NC, N_PAD, 16)[:, :N]
    m1 = _mm_tc(x, W1)
    hp1A, hp1B, dis = _dis_hp_tc(deg_parts[0], deg_parts[1], m1)

    s1 = _agg_sc(hp1A, hp1B, src, dst, zeros64).reshape(2, NC, N_PAD, HALF)[:, :, :N]
    hp2A, hp2B = _mid_tc(s1[0, 0], s1[0, 1], s1[1, 0], s1[1, 1],
                         hp1A, hp1B, dis, b1r, Wcat[:HALF], Wcat[HALF:])

    s2 = _agg_sc(hp2A, hp2B, src, dst, zeros64).reshape(2, NC, N_PAD, HALF)[:, :, :N]
    mu, logstd = _final_tc(s2[0, 0], s2[0, 1], s2[1, 0], s2[1, 1],
                           hp2A, hp2B, dis, bmur, blsr)
    return (mu, logstd)


# speedup vs baseline: 3.2918x; 1.2474x over previous
import functools

import jax
import jax.numpy as jnp
from jax import lax
from jax.experimental import pallas as pl
from jax.experimental.pallas import tpu as pltpu
from jax.experimental.pallas import tpu_sc as plsc

N = 10000
E = 320000
IN_CH = 128
OUT_CH = 64
CAT = 2 * OUT_CH
HALF = 64

NC = 2
NS = 16
NW = NC * NS
EPT = E // NW
K = 80
NCHUNK = 128
EPT_PAD = NCHUNK * K
NBUF = 4
NGRP = NCHUNK // NBUF
N_PAD = 10240
RPT = N_PAD // NS
RCH = 160
NRCH = RPT // RCH


def _sc_mesh():
    return plsc.VectorSubcoreMesh(core_axis_name="c", subcore_axis_name="s")


def _deg_sc(dst_idx, ones_blk, zeros_blk):
    @functools.partial(
        pl.kernel,
        out_type=jax.ShapeDtypeStruct((NC, NS, RPT, 16), jnp.float32),
        mesh=_sc_mesh(),
        compiler_params=pltpu.CompilerParams(use_tc_tiling_on_sc=False),
        scratch_types=[
            pltpu.VMEM((NCHUNK, K), jnp.int32),
            pltpu.VMEM((K, 16), jnp.float32),
            pltpu.VMEM((RCH, 16), jnp.float32),
            pltpu.VMEM_SHARED((N_PAD, 16), jnp.float32),
        ],
    )
    def body(dst_hbm, ones_hbm, zeros_hbm, out_hbm, idx_v, ones_v, buf_v, acc_sh):
        c = lax.axis_index("c")
        s = lax.axis_index("s")
        w = c * NS + s
        pltpu.sync_copy(zeros_hbm, buf_v)
        for i in range(NRCH):
            pltpu.sync_copy(buf_v, acc_sh.at[pl.ds(s * RPT + i * RCH, RCH)])
        pltpu.sync_copy(ones_hbm, ones_v)
        pltpu.sync_copy(dst_hbm.at[w], idx_v)
        plsc.subcore_barrier()

        def chunk(j, carry):
            pltpu.sync_copy(ones_v, acc_sh.at[idx_v.at[j]], add=True)
            return carry

        lax.fori_loop(0, NCHUNK, chunk, 0)
        plsc.subcore_barrier()
        for i in range(NRCH):
            pltpu.sync_copy(acc_sh.at[pl.ds(s * RPT + i * RCH, RCH)], buf_v)
            pltpu.sync_copy(buf_v, out_hbm.at[c, s, pl.ds(i * RCH, RCH)])

    return body(dst_idx, ones_blk, zeros_blk)


def _agg_sc(hpA, hpB, src_idx, dst_idx, zeros_blk):
    @functools.partial(
        pl.kernel,
        out_type=jax.ShapeDtypeStruct((2, NC, NS, RPT, HALF), jnp.float32),
        mesh=_sc_mesh(),
        compiler_params=pltpu.CompilerParams(use_tc_tiling_on_sc=False),
        scratch_types=[
            pltpu.VMEM((NCHUNK, K), jnp.int32),
            pltpu.VMEM((NCHUNK, K), jnp.int32),
            [pltpu.VMEM((K, HALF), jnp.float32)] * NBUF,
            pltpu.VMEM((RCH, HALF), jnp.float32),
            pltpu.VMEM_SHARED((N_PAD, HALF), jnp.float32),
            [pltpu.SemaphoreType.DMA] * NBUF,
        ],
    )
    def body(hpA_hbm, hpB_hbm, src_hbm, dst_hbm, zeros_hbm, out_hbm,
             sidx_v, didx_v, rows_v, buf_v, acc_sh, sems):
        c = lax.axis_index("c")
        s = lax.axis_index("s")
        w = c * NS + s
        pltpu.sync_copy(src_hbm.at[w], sidx_v)
        pltpu.sync_copy(dst_hbm.at[w], didx_v)

        for half, hp_hbm in ((0, hpA_hbm), (1, hpB_hbm)):
            pltpu.sync_copy(zeros_hbm, buf_v)
            for i in range(NRCH):
                pltpu.sync_copy(buf_v, acc_sh.at[pl.ds(s * RPT + i * RCH, RCH)])
            plsc.subcore_barrier()

            for b in range(NBUF):
                pltpu.async_copy(hp_hbm.at[sidx_v.at[b]], rows_v[b], sems[b])

            def grp(p, carry):
                for b in range(NBUF):
                    j = NBUF * p + b
                    pltpu.make_async_copy(hp_hbm.at[sidx_v.at[j]], rows_v[b],
                                          sems[b]).wait()
                    pltpu.sync_copy(rows_v[b], acc_sh.at[didx_v.at[j]],
                                    add=True)
                    pltpu.async_copy(hp_hbm.at[sidx_v.at[j + NBUF]],
                                     rows_v[b], sems[b])
                return carry

            lax.fori_loop(0, NGRP - 1, grp, 0)
            for b in range(NBUF):
                j = NBUF * (NGRP - 1) + b
                pltpu.make_async_copy(hp_hbm.at[sidx_v.at[j]], rows_v[b],
                                      sems[b]).wait()
                pltpu.sync_copy(rows_v[b], acc_sh.at[didx_v.at[j]], add=True)
            plsc.subcore_barrier()
            for i in range(NRCH):
                pltpu.sync_copy(acc_sh.at[pl.ds(s * RPT + i * RCH, RCH)], buf_v)
                pltpu.sync_copy(buf_v, out_hbm.at[half, c, s, pl.ds(i * RCH, RCH)])
            plsc.subcore_barrier()

    return body(hpA, hpB, src_idx, dst_idx, zeros_blk)


_BLK = 1000
_GRID = N // _BLK


def _mm_tc(x, W):
    def body(x_ref, w_ref, o_ref):
        o_ref[...] = jnp.dot(x_ref[...], w_ref[...],
                             preferred_element_type=jnp.float32)

    return pl.pallas_call(
        body,
        grid=(_GRID,),
        in_specs=[
            pl.BlockSpec((_BLK, x.shape[1]), lambda i: (i, 0)),
            pl.BlockSpec(W.shape, lambda i: (0, 0)),
        ],
        out_specs=pl.BlockSpec((_BLK, W.shape[1]), lambda i: (i, 0)),
        out_shape=jax.ShapeDtypeStruct((N, W.shape[1]), jnp.float32),
    )(x, W)


def _dis_hp_tc(deg0, deg1, m1):
    def body(d0_ref, d1_ref, m_ref, hpA_ref, hpB_ref, dis_ref):
        deg = d0_ref[:, :1] + d1_ref[:, :1] + 1.0
        dis = lax.rsqrt(deg)
        dis_ref[...] = dis
        hp = dis * m_ref[...]
        hpA_ref[...] = hp[:, :HALF]
        hpB_ref[...] = hp[:, HALF:]

    return pl.pallas_call(
        body,
        grid=(_GRID,),
        in_specs=[
            pl.BlockSpec((_BLK, 16), lambda i: (i, 0)),
            pl.BlockSpec((_BLK, 16), lambda i: (i, 0)),
            pl.BlockSpec((_BLK, CAT), lambda i: (i, 0)),
        ],
        out_specs=[
            pl.BlockSpec((_BLK, HALF), lambda i: (i, 0)),
            pl.BlockSpec((_BLK, HALF), lambda i: (i, 0)),
            pl.BlockSpec((_BLK, 1), lambda i: (i, 0)),
        ],
        out_shape=[
            jax.ShapeDtypeStruct((N, HALF), jnp.float32),
            jax.ShapeDtypeStruct((N, HALF), jnp.float32),
            jax.ShapeDtypeStruct((N, 1), jnp.float32),
        ],
    )(deg0, deg1, m1)


def _mid_tc(p0A, p1A, p0B, p1B, hp1A, hp1B, dis, b1, WcatA, WcatB):
    def body(p0A_ref, p1A_ref, p0B_ref, p1B_ref, hpA_ref, hpB_ref,
             dis_ref, b_ref, wA_ref, wB_ref, oA_ref, oB_ref):
        dis = dis_ref[...]
        tA = dis * (p0A_ref[...] + p1A_ref[...] + hpA_ref[...]) + b_ref[:, :HALF]
        tB = dis * (p0B_ref[...] + p1B_ref[...] + hpB_ref[...]) + b_ref[:, HALF:]
        hA = jnp.maximum(tA, 0.0)
        hB = jnp.maximum(tB, 0.0)
        nrm = jnp.sqrt(jnp.sum(hA * hA, axis=1, keepdims=True)
                       + jnp.sum(hB * hB, axis=1, keepdims=True))
        inv = 1.0 / jnp.maximum(nrm, 1e-12)
        hA = hA * inv
        hB = hB * inv
        hA = jnp.maximum(hA, 0.0)
        hB = jnp.maximum(hB, 0.0)
        m2 = (jnp.dot(hA, wA_ref[...], preferred_element_type=jnp.float32)
              + jnp.dot(hB, wB_ref[...], preferred_element_type=jnp.float32))
        hp2 = dis * m2
        oA_ref[...] = hp2[:, :HALF]
        oB_ref[...] = hp2[:, HALF:]

    blk = lambda d: pl.BlockSpec((_BLK, d), lambda i: (i, 0))
    return pl.pallas_call(
        body,
        grid=(_GRID,),
        in_specs=[
            blk(HALF), blk(HALF), blk(HALF), blk(HALF), blk(HALF), blk(HALF),
            blk(1),
            pl.BlockSpec((1, CAT), lambda i: (0, 0)),
            pl.BlockSpec((HALF, CAT), lambda i: (0, 0)),
            pl.BlockSpec((HALF, CAT), lambda i: (0, 0)),
        ],
        out_specs=[blk(HALF), blk(HALF)],
        out_shape=[
            jax.ShapeDtypeStruct((N, HALF), jnp.float32),
            jax.ShapeDtypeStruct((N, HALF), jnp.float32),
        ],
    )(p0A, p1A, p0B, p1B, hp1A, hp1B, dis, b1, WcatA, WcatB)


def _final_tc(p0A, p1A, p0B, p1B, hp2A, hp2B, dis, bmu, bls):
    def body(p0A_ref, p1A_ref, p0B_ref, p1B_ref, hpA_ref, hpB_ref,
             dis_ref, bmu_ref, bls_ref, mu_ref, ls_ref):
        dis = dis_ref[...]
        mu_ref[...] = dis * (p0A_ref[...] + p1A_ref[...] + hpA_ref[...]) \
            + bmu_ref[...]
        ls_ref[...] = dis * (p0B_ref[...] + p1B_ref[...] + hpB_ref[...]) \
            + bls_ref[...]

    blk = lambda d: pl.BlockSpec((_BLK, d), lambda i: (i, 0))
    return pl.pallas_call(
        body,
        grid=(_GRID,),
        in_specs=[
            blk(HALF), blk(HALF), blk(HALF), blk(HALF), blk(HALF), blk(HALF),
            blk(1),
            pl.BlockSpec((1, HALF), lambda i: (0, 0)),
            pl.BlockSpec((1, HALF), lambda i: (0, 0)),
        ],
        out_specs=[blk(HALF), blk(HALF)],
        out_shape=[
            jax.ShapeDtypeStruct((N, HALF), jnp.float32),
            jax.ShapeDtypeStruct((N, HALF), jnp.float32),
        ],
    )(p0A, p1A, p0B, p1B, hp2A, hp2B, dis, bmu, bls)


def kernel(x, edge_index, W1, b1, Wmu, bmu, Wls, bls):
    n_fake = EPT_PAD - EPT
    fake_src = jnp.broadcast_to(
        (jnp.arange(n_fake, dtype=jnp.int32) * 41) % N, (NW, n_fake))
    fake_dst = jnp.broadcast_to(
        N + jnp.arange(n_fake, dtype=jnp.int32), (NW, n_fake))
    src = jnp.concatenate(
        [edge_index[0].astype(jnp.int32).reshape(NW, EPT), fake_src], axis=1)
    dst = jnp.concatenate(
        [edge_index[1].astype(jnp.int32).reshape(NW, EPT), fake_dst], axis=1)
    src = src.reshape(NW, NCHUNK, K)
    dst = dst.reshape(NW, NCHUNK, K)

    ones_blk = jnp.ones((K, 16), jnp.float32)
    zeros16 = jnp.zeros((RCH, 16), jnp.float32)
    zeros64 = jnp.zeros((RCH, HALF), jnp.float32)

    Wcat = jnp.concatenate([Wmu, Wls], axis=1)
    b1r = b1.reshape(1, CAT)
    bmur = bmu.reshape(1, HALF)
    blsr = bls.reshape(1, HALF)

    deg_parts = _deg_sc(dst, ones_blk, zeros16).reshape(NC, N_PAD, 16)[:, :N]
    m1 = _mm_tc(x, W1)
    hp1A, hp1B, dis = _dis_hp_tc(deg_parts[0], deg_parts[1], m1)

    s1 = _agg_sc(hp1A, hp1B, src, dst, zeros64).reshape(2, NC, N_PAD, HALF)[:, :, :N]
    hp2A, hp2B = _mid_tc(s1[0, 0], s1[0, 1], s1[1, 0], s1[1, 1],
                         hp1A, hp1B, dis, b1r, Wcat[:HALF], Wcat[HALF:])

    s2 = _agg_sc(hp2A, hp2B, src, dst, zeros64).reshape(2, NC, N_PAD, HALF)[:, :, :N]
    mu, logstd = _final_tc(s2[0, 0], s2[0, 1], s2[1, 0], s2[1, 1],
                           hp2A, hp2B, dis, bmur, blsr)
    return (mu, logstd)
